# baseline (speedup 1.0000x reference)
import functools

import jax
import jax.numpy as jnp
from jax import lax
from jax.experimental import pallas as pl
from jax.experimental.pallas import tpu as pltpu
from jax.experimental.pallas import tpu_sc as plsc

BATCH = 16384
DIM = 16
NC = 2
NS = 16
NW = NC * NS
BPW = BATCH // NW
NSLICE = BPW // 16

_mesh = plsc.VectorSubcoreMesh(core_axis_name="c", subcore_axis_name="s")


@functools.partial(
    pl.kernel,
    mesh=_mesh,
    out_type=jax.ShapeDtypeStruct((NW, 16), jnp.float32),
    compiler_params=pltpu.CompilerParams(
        needs_layout_passes=False,
        use_tc_tiling_on_sc=False,
    ),
    scratch_types=[
        pltpu.VMEM((BPW,), jnp.int32),
        pltpu.VMEM((BPW,), jnp.int32),
        pltpu.VMEM((BPW,), jnp.int32),
        pltpu.VMEM((DIM, BPW), jnp.float32),
        pltpu.VMEM((DIM, BPW), jnp.float32),
        pltpu.VMEM((DIM, BPW), jnp.float32),
        pltpu.VMEM((16,), jnp.float32),
        pltpu.SemaphoreType.DMA,
        pltpu.SemaphoreType.DMA,
        pltpu.SemaphoreType.DMA,
        pltpu.SemaphoreType.DMA,
    ],
)
def _sc_score(users_hbm, pos_hbm, neg_hbm, ut_hbm, it_hbm, out_hbm,
              iu, ip, ineg, ru, rp, rn, accv, si, su, sp_sem, sn_sem):
    wid = lax.axis_index("s") * NC + lax.axis_index("c")
    base = wid * BPW

    idx_copies = [
        pltpu.async_copy(users_hbm.at[pl.ds(base, BPW)], iu, si),
        pltpu.async_copy(pos_hbm.at[pl.ds(base, BPW)], ip, si),
        pltpu.async_copy(neg_hbm.at[pl.ds(base, BPW)], ineg, si),
    ]
    for c in idx_copies:
        c.wait()

    copies = []
    for d in range(DIM):
        copies.append(pltpu.async_copy(ut_hbm.at[d].at[iu], ru.at[d], su))
        copies.append(pltpu.async_copy(it_hbm.at[d].at[ip], rp.at[d], sp_sem))
        copies.append(pltpu.async_copy(it_hbm.at[d].at[ineg], rn.at[d], sn_sem))
    for c in copies:
        c.wait()

    zero = jnp.zeros((16,), jnp.float32)

    def slice_step(s, acc):
        col = pl.ds(s * 16, 16)
        sp = zero
        sn = zero
        for d in range(DIM):
            uc = ru[d, col]
            sp = sp + uc * rp[d, col]
            sn = sn + uc * rn[d, col]
        acc = acc + 1.0 / (1.0 + jnp.exp(-sp)) + 1.0 / (1.0 + jnp.exp(sn))
        return acc

    acc = lax.fori_loop(0, NSLICE, slice_step, zero)
    accv[...] = acc
    pltpu.sync_copy(accv, out_hbm.at[wid])


def _tc_reduce_body(p_ref, o_ref):
    o_ref[...] = (-jnp.sum(p_ref[...]) / BATCH).reshape(1, 1)


_tc_reduce = pl.pallas_call(
    _tc_reduce_body,
    out_shape=jax.ShapeDtypeStruct((1, 1), jnp.float32),
)


def kernel(users, pos_items, neg_items, user_table, item_table):
    u = users.astype(jnp.int32)
    p = pos_items.astype(jnp.int32)
    n = neg_items.reshape(-1).astype(jnp.int32)
    partials = _sc_score(u, p, n, user_table.T, item_table.T)
    loss = _tc_reduce(partials)[0, 0]
    return (loss, loss, jnp.float32(0.0))


# speedup vs baseline: 18.2766x; 18.2766x over previous
import functools

import jax
import jax.numpy as jnp
from jax import lax
from jax.experimental import pallas as pl
from jax.experimental.pallas import tpu as pltpu
from jax.experimental.pallas import tpu_sc as plsc

BATCH = 16384
DIM = 16
NROWS = 1000000
PADC = 1048576
CHUNK = 65536
NCHUNK = PADC // CHUNK
NC = 2
NS = 16
NW = NC * NS
BPW = BATCH // NW
NSLICE = BPW // 16


def _relayout_body(t_ref, *o_refs):
    for d in range(DIM):
        o_refs[d][...] = t_ref[d, :]


_relayout = pl.pallas_call(
    _relayout_body,
    grid=(NCHUNK,),
    in_specs=[pl.BlockSpec((DIM, CHUNK), lambda c: (0, c))],
    out_specs=tuple(
        pl.BlockSpec((CHUNK,), lambda c: (c,)) for _ in range(DIM)
    ),
    out_shape=tuple(
        jax.ShapeDtypeStruct((PADC,), jnp.float32) for _ in range(DIM)
    ),
)


_mesh = plsc.VectorSubcoreMesh(core_axis_name="c", subcore_axis_name="s")


@functools.partial(
    pl.kernel,
    mesh=_mesh,
    out_type=jax.ShapeDtypeStruct((NW, 16), jnp.float32),
    compiler_params=pltpu.CompilerParams(
        needs_layout_passes=False,
        use_tc_tiling_on_sc=False,
    ),
    scratch_types=[
        pltpu.VMEM((BPW,), jnp.int32),
        pltpu.VMEM((BPW,), jnp.int32),
        pltpu.VMEM((BPW,), jnp.int32),
        pltpu.VMEM((DIM, BPW), jnp.float32),
        pltpu.VMEM((DIM, BPW), jnp.float32),
        pltpu.VMEM((DIM, BPW), jnp.float32),
        pltpu.VMEM((16,), jnp.float32),
        pltpu.SemaphoreType.DMA,
        pltpu.SemaphoreType.DMA,
        pltpu.SemaphoreType.DMA,
        pltpu.SemaphoreType.DMA,
    ],
)
def _sc_score(users_hbm, pos_hbm, neg_hbm, *rest):
    ut_rows = rest[:DIM]
    it_rows = rest[DIM:2 * DIM]
    out_hbm = rest[2 * DIM]
    (iu, ip, ineg, ru, rp, rn, accv, si, su, sp_sem, sn_sem) = rest[2 * DIM + 1:]
    wid = lax.axis_index("s") * NC + lax.axis_index("c")
    base = wid * BPW

    idx_copies = [
        pltpu.async_copy(users_hbm.at[pl.ds(base, BPW)], iu, si),
        pltpu.async_copy(pos_hbm.at[pl.ds(base, BPW)], ip, si),
        pltpu.async_copy(neg_hbm.at[pl.ds(base, BPW)], ineg, si),
    ]
    for c in idx_copies:
        c.wait()

    copies = []
    for d in range(DIM):
        copies.append(pltpu.async_copy(ut_rows[d].at[iu], ru.at[d], su))
        copies.append(pltpu.async_copy(it_rows[d].at[ip], rp.at[d], sp_sem))
        copies.append(pltpu.async_copy(it_rows[d].at[ineg], rn.at[d], sn_sem))
    for c in copies:
        c.wait()

    zero = jnp.zeros((16,), jnp.float32)

    def slice_step(s, acc):
        col = pl.ds(s * 16, 16)
        sp = zero
        sn = zero
        for d in range(DIM):
            uc = ru[d, col]
            sp = sp + uc * rp[d, col]
            sn = sn + uc * rn[d, col]
        acc = acc + 1.0 / (1.0 + jnp.exp(-sp)) + 1.0 / (1.0 + jnp.exp(sn))
        return acc

    acc = lax.fori_loop(0, NSLICE, slice_step, zero)
    accv[...] = acc
    pltpu.sync_copy(accv, out_hbm.at[wid])


def _tc_reduce_body(p_ref, o_ref):
    o_ref[...] = (-jnp.sum(p_ref[...]) / BATCH).reshape(1, 1)


_tc_reduce = pl.pallas_call(
    _tc_reduce_body,
    out_shape=jax.ShapeDtypeStruct((1, 1), jnp.float32),
)


def kernel(users, pos_items, neg_items, user_table, item_table):
    u = users.astype(jnp.int32)
    p = pos_items.astype(jnp.int32)
    n = neg_items.reshape(-1).astype(jnp.int32)
    ut_rows = _relayout(user_table.T)
    it_rows = _relayout(item_table.T)
    partials = _sc_score(u, p, n, *ut_rows, *it_rows)
    loss = _tc_reduce(partials)[0, 0]
    return (loss, loss, jnp.float32(0.0))
